# baseline (speedup 1.0000x reference)
import jax
import jax.numpy as jnp
from jax import lax
from jax.experimental import pallas as pl
from jax.experimental.pallas import tpu as pltpu
from jax.experimental.pallas import tpu_sc as plsc

B = 64
H = 512
W = 512
N = H * W
K = 1 + round(0.01 * 99.0 * (N - 1))

NC = 2
NS = 16
NW = NC * NS
SPT = B // NW

NB = 4096
NB_C = 128
L = 16
CH = 16384
RPC = CH // W
NCH = N // CH
VPR = W // L


def _tc_body(pred_ref, tgt_ref, mask_ref, loss_ref, sums_ref):
    lv = mask_ref[...] * lax.abs(pred_ref[...] - tgt_ref[...])
    loss_ref[...] = lv
    sums_ref[...] = jnp.full((1, 1, 128), jnp.sum(lv), jnp.float32)


def _tc_loss(pred, tgt, mask):
    return pl.pallas_call(
        _tc_body,
        grid=(B,),
        in_specs=[
            pl.BlockSpec((1, H, W), lambda b: (b, 0, 0)),
            pl.BlockSpec((1, H, W), lambda b: (b, 0, 0)),
            pl.BlockSpec((1, H, W), lambda b: (b, 0, 0)),
        ],
        out_specs=[
            pl.BlockSpec((1, H, W), lambda b: (b, 0, 0)),
            pl.BlockSpec((1, 1, 128), lambda b: (b, 0, 0)),
        ],
        out_shape=[
            jax.ShapeDtypeStruct((B, H, W), jnp.float32),
            jax.ShapeDtypeStruct((B, 1, 128), jnp.float32),
        ],
    )(pred, tgt, mask)


def _zero_hist(hist, nb):
    def body(i, _):
        hist[pl.ds(i * L, L)] = jnp.zeros((L,), jnp.int32)
        return 0
    lax.fori_loop(0, nb, body, 0)


def _select(hist, tmp, tmp2, r, nb):
    lane = lax.broadcasted_iota(jnp.int32, (L,), 0)
    zvec = jnp.zeros((L,), jnp.int32)
    nblk = nb // L

    def p1(j, _):
        v = zvec
        for m in range(L):
            v = v + hist[pl.ds(j * (L * L) + m * L, L)]
        tot = jnp.sum(v)
        plsc.store_scatter(tmp2, [jnp.full((L,), j, jnp.int32)],
                           jnp.full((L,), tot, jnp.int32),
                           mask=lane == 0)
        return 0

    lax.fori_loop(0, nblk, p1, 0)

    zero = jnp.int32(0)
    nv2 = (nblk + L - 1) // L
    ones_v = jnp.ones((L,), jnp.int32)

    def p2(jj, carry):
        cum, bblk, cumbef = carry
        t = tmp2[pl.ds(jj * L, L)]
        t = jnp.where((jj * L + lane) < nblk, t, zvec)
        cv = plsc.cumsum(t) + cum
        mlt = cv < r
        bblk = bblk + jnp.sum(jnp.where(mlt, ones_v, zvec))
        cumbef = cumbef + jnp.sum(jnp.where(mlt, t, zvec))
        cum = cum + jnp.sum(t)
        return (cum, bblk, cumbef)

    _cum, jb, cumbef = lax.fori_loop(0, nv2, p2, (zero, zero, zero))

    def p3(m, carry):
        cum, bstar, cumbef_, cnt = carry
        c_m = jnp.sum(hist[pl.ds(jb * (L * L) + m * L, L)])
        newcum = cum + c_m
        is_lt = newcum < r
        bstar = jnp.where(is_lt, bstar + 1, bstar)
        cumbef_ = jnp.where(is_lt, cumbef_ + c_m, cumbef_)
        issel = jnp.logical_and(newcum >= r, cum < r)
        cnt = jnp.where(issel, c_m, cnt)
        return (newcum, bstar, cumbef_, cnt)

    _c, bstar, cumbef, cnt = lax.fori_loop(
        0, L, p3, (cumbef, jb * L, cumbef, zero))

    _zero_hist(hist, nb)
    return bstar, r - cumbef, cnt


def _stream_pass(src_hbm, s, buf0, buf1, sem0, sem1, row_fn):
    def start(c, buf, sem):
        off = pl.multiple_of(c * RPC, RPC)
        pltpu.async_copy(src_hbm.at[s, pl.ds(off, RPC), :], buf, sem)

    def wait(c, buf, sem):
        off = pl.multiple_of(c * RPC, RPC)
        pltpu.make_async_copy(src_hbm.at[s, pl.ds(off, RPC), :], buf,
                              sem).wait()

    def process(buf):
        def rbody(r, _):
            row_fn(buf, r)
            return 0
        lax.fori_loop(0, RPC, rbody, 0)

    start(0, buf0, sem0)
    start(1, buf1, sem1)

    def body(c2, _):
        c0 = c2 * 2
        wait(c0, buf0, sem0)
        process(buf0)

        @pl.when(c0 + 2 < NCH)
        def _():
            start(c0 + 2, buf0, sem0)

        wait(c0 + 1, buf1, sem1)
        process(buf1)

        @pl.when(c0 + 3 < NCH)
        def _():
            start(c0 + 3, buf1, sem1)

        return 0

    lax.fori_loop(0, NCH // 2, body, 0)


def _sc_body(loss_hbm, qbits_hbm,
             buf0, buf1, hist, tmp, tmp2, outbuf_i, sem0, sem1):
    wid = lax.axis_index("s") * NC + lax.axis_index("c")
    lane = lax.broadcasted_iota(jnp.int32, (L,), 0)
    ones_i = jnp.ones((L,), jnp.int32)

    with jax.named_scope("zero_init"):
        _zero_hist(hist, NB)

    results = []
    for local in range(SPT):
        s = wid * SPT + local

        def row_a(buf, r):
            for u in range(VPR):
                bits = lax.bitcast_convert_type(buf[r, pl.ds(u * L, L)],
                                                jnp.int32)
                d = lax.shift_right_logical(bits, 19)
                plsc.addupdate_scatter(hist, [lax.shift_left(d, 4) + lane],
                                       ones_i)

        with jax.named_scope("stream_l1"):
            _stream_pass(loss_hbm, s, buf0, buf1, sem0, sem1, row_a)
        with jax.named_scope("sel_l1"):
            b1, r2, _c1 = _select(hist, tmp, tmp2, jnp.int32(K), NB)

        def row_b(buf, r):
            for u in range(VPR):
                bits = lax.bitcast_convert_type(buf[r, pl.ds(u * L, L)],
                                                jnp.int32)
                match = lax.shift_right_logical(bits, 19) == b1
                d = lax.bitwise_and(lax.shift_right_logical(bits, 7),
                                    jnp.int32(0xFFF))
                plsc.addupdate_scatter(hist, [lax.shift_left(d, 4) + lane],
                                       ones_i, mask=match)

        with jax.named_scope("stream_l2"):
            _stream_pass(loss_hbm, s, buf0, buf1, sem0, sem1, row_b)
        with jax.named_scope("sel_l2"):
            b2, r3, _c2 = _select(hist, tmp, tmp2, r2, NB)
        pref = b1 * 4096 + b2

        def row_c(buf, r):
            for u in range(VPR):
                bits = lax.bitcast_convert_type(buf[r, pl.ds(u * L, L)],
                                                jnp.int32)
                match = lax.shift_right_logical(bits, 7) == pref
                d = lax.bitwise_and(bits, jnp.int32(0x7F))
                plsc.addupdate_scatter(hist, [lax.shift_left(d, 4) + lane],
                                       ones_i, mask=match)

        with jax.named_scope("stream_l3"):
            _stream_pass(loss_hbm, s, buf0, buf1, sem0, sem1, row_c)
        with jax.named_scope("sel_l3"):
            b3, _r4, _c3 = _select(hist, tmp, tmp2, r3, NB_C)
        results.append(pref * 128 + b3)

    q0, q1 = results
    row_i = jnp.where(lane == 0, jnp.full((L,), q0, jnp.int32),
                      jnp.where(lane == 1, jnp.full((L,), q1, jnp.int32),
                                jnp.zeros((L,), jnp.int32)))
    outbuf_i[...] = row_i
    pltpu.sync_copy(outbuf_i, qbits_hbm.at[wid])


@jax.jit
def kernel(predicted, target, mask):
    pred3 = predicted.reshape(B, H, W)
    tgt3 = target.reshape(B, H, W)
    mask3 = mask.reshape(B, H, W)

    loss, sums = _tc_loss(pred3, tgt3, mask3)

    mesh = plsc.VectorSubcoreMesh(core_axis_name="c", subcore_axis_name="s",
                                  num_cores=NC, num_subcores=NS)
    qbits = pl.kernel(
        _sc_body,
        out_type=jax.ShapeDtypeStruct((NW, L), jnp.int32),
        mesh=mesh,
        compiler_params=pltpu.CompilerParams(needs_layout_passes=False),
        scratch_types=[
            pltpu.VMEM((RPC, W), jnp.float32),
            pltpu.VMEM((RPC, W), jnp.float32),
            pltpu.VMEM((NB * L,), jnp.int32),
            pltpu.VMEM((NB,), jnp.int32),
            pltpu.VMEM((NB // L,), jnp.int32),
            pltpu.VMEM((L,), jnp.int32),
            pltpu.SemaphoreType.DMA,
            pltpu.SemaphoreType.DMA,
        ],
    )(loss)

    q_loss = lax.bitcast_convert_type(qbits[:, :SPT].reshape(B), jnp.float32)
    wmae = jnp.sum(sums[:, 0, 0]) / (B * N)
    return (q_loss, wmae)


# speedup vs baseline: 1.1335x; 1.1335x over previous
import jax
import jax.numpy as jnp
from jax import lax
from jax.experimental import pallas as pl
from jax.experimental.pallas import tpu as pltpu
from jax.experimental.pallas import tpu_sc as plsc

B = 64
H = 512
W = 512
N = H * W
K = 1 + round(0.01 * 99.0 * (N - 1))

NC = 2
NS = 16
NW = NC * NS
SPT = B // NW

NB = 4096
NB_C = 128
L = 16
CH = 16384
RPC = CH // W
NCH = N // CH
VPR = W // L


def _tc_body(pred_ref, tgt_ref, mask_ref, loss_ref, sums_ref):
    lv = mask_ref[...] * lax.abs(pred_ref[...] - tgt_ref[...])
    loss_ref[...] = lv
    sums_ref[...] = jnp.full((1, 1, 128), jnp.sum(lv), jnp.float32)


def _tc_loss(pred, tgt, mask):
    return pl.pallas_call(
        _tc_body,
        grid=(B,),
        in_specs=[
            pl.BlockSpec((1, H, W), lambda b: (b, 0, 0)),
            pl.BlockSpec((1, H, W), lambda b: (b, 0, 0)),
            pl.BlockSpec((1, H, W), lambda b: (b, 0, 0)),
        ],
        out_specs=[
            pl.BlockSpec((1, H, W), lambda b: (b, 0, 0)),
            pl.BlockSpec((1, 1, 128), lambda b: (b, 0, 0)),
        ],
        out_shape=[
            jax.ShapeDtypeStruct((B, H, W), jnp.float32),
            jax.ShapeDtypeStruct((B, 1, 128), jnp.float32),
        ],
    )(pred, tgt, mask)


def _select(tmp, tmp2, r, nb):
    lane = lax.broadcasted_iota(jnp.int32, (L,), 0)
    zvec = jnp.zeros((L,), jnp.int32)
    ones_v = jnp.ones((L,), jnp.int32)
    nblk = nb // L

    def p2a(j, _):
        t = tmp[pl.ds(j * L, L)]
        tot = jnp.sum(t)
        plsc.store_scatter(tmp2, [jnp.full((L,), j, jnp.int32)],
                           jnp.full((L,), tot, jnp.int32),
                           mask=lane == 0)
        return 0

    lax.fori_loop(0, nblk, p2a, 0)

    zero = jnp.int32(0)
    nv2 = (nblk + L - 1) // L

    def p2b(jj, carry):
        cum, bblk, cumbef = carry
        t = tmp2[pl.ds(jj * L, L)]
        t = jnp.where((jj * L + lane) < nblk, t, zvec)
        cv = plsc.cumsum(t) + cum
        mlt = cv < r
        bblk = bblk + jnp.sum(jnp.where(mlt, ones_v, zvec))
        cumbef = cumbef + jnp.sum(jnp.where(mlt, t, zvec))
        cum = cum + jnp.sum(t)
        return (cum, bblk, cumbef)

    _cum, jb, cumbef = lax.fori_loop(0, nv2, p2b, (zero, zero, zero))

    v = tmp[pl.ds(jb * L, L)]
    cv = plsc.cumsum(v) + cumbef
    mlt = cv < r
    msel = jnp.logical_and(cv >= r, (cv - v) < r)
    loff = jnp.sum(jnp.where(mlt, ones_v, zvec))
    cumbef = cumbef + jnp.sum(jnp.where(mlt, v, zvec))
    cnt = jnp.sum(jnp.where(msel, v, zvec))
    return jb * L + loff, r - cumbef, cnt


def _sc_body(loss_hbm, qbits_hbm,
             buf0, buf1, idx0, idx1, val0, val1, tmp, tmp2, zb, outbuf_i,
             hist_sh, sem0, sem1, ssem0, ssem1):
    cid = lax.axis_index("c")
    sid = lax.axis_index("s")
    wid = sid * NC + cid
    lane = lax.broadcasted_iota(jnp.int32, (L,), 0)
    ones_i = jnp.ones((L,), jnp.int32)
    zeros_i = jnp.zeros((L,), jnp.int32)

    bufs = (buf0, buf1)
    idxs = (idx0, idx1)
    vals = (val0, val1)
    sems = (sem0, sem1)
    ssems = (ssem0, ssem1)

    def zb_init(i, _):
        zb[pl.ds(i * L, L)] = zeros_i
        return 0
    lax.fori_loop(0, NB // L, zb_init, 0)

    def ones_fill(i, _):
        val0[pl.ds(i * L, L)] = ones_i
        val1[pl.ds(i * L, L)] = ones_i
        return 0
    lax.fori_loop(0, CH // L, ones_fill, 0)

    for local in range(SPT):
        rb0 = (sid * SPT + local) * NB
        pltpu.sync_copy(zb, hist_sh.at[pl.ds(rb0, NB)])

    def stream_count(s, rb, vreg_fn, use_val):
        def start_in(c, p):
            off = pl.multiple_of(c * RPC, RPC)
            pltpu.async_copy(loss_hbm.at[s, pl.ds(off, RPC), :], bufs[p],
                             sems[p])

        def wait_in(c, p):
            off = pl.multiple_of(c * RPC, RPC)
            pltpu.make_async_copy(loss_hbm.at[s, pl.ds(off, RPC), :],
                                  bufs[p], sems[p]).wait()

        def fire_scatter(p):
            pltpu.async_copy(vals[p], hist_sh.at[idxs[p]], ssems[p],
                             add=True)

        def drain_scatter(p):
            pltpu.make_async_copy(vals[p], hist_sh.at[idxs[p]],
                                  ssems[p]).wait()

        def compute(p):
            buf, idx, val = bufs[p], idxs[p], vals[p]

            def rbody(r, _):
                base = r * W
                for u in range(VPR):
                    bits = lax.bitcast_convert_type(
                        buf[r, pl.ds(u * L, L)], jnp.int32)
                    d, v = vreg_fn(bits)
                    idx[pl.ds(base + u * L, L)] = d + rb
                    if use_val:
                        val[pl.ds(base + u * L, L)] = v
                return 0

            lax.fori_loop(0, RPC, rbody, 0)

        start_in(0, 0)
        start_in(1, 1)

        def body(c2, _):
            c0 = c2 * 2
            for p in range(2):
                c = c0 + p
                wait_in(c, p)

                @pl.when(c2 > 0)
                def _():
                    drain_scatter(p)

                compute(p)
                fire_scatter(p)

                @pl.when(c + 2 < NCH)
                def _():
                    start_in(c + 2, p)
            return 0

        lax.fori_loop(0, NCH // 2, body, 0)
        drain_scatter(0)
        drain_scatter(1)

    results = []
    for local in range(SPT):
        s = wid * SPT + local
        rb = (sid * SPT + local) * NB

        def read_and_rezero(nb):
            pltpu.sync_copy(hist_sh.at[pl.ds(rb, nb)], tmp.at[pl.ds(0, nb)])
            pltpu.sync_copy(zb.at[pl.ds(0, nb)], hist_sh.at[pl.ds(rb, nb)])

        def f1(bits):
            return lax.shift_right_logical(bits, 19), None

        stream_count(s, rb, f1, use_val=False)
        read_and_rezero(NB)
        b1, r2, _c1 = _select(tmp, tmp2, jnp.int32(K), NB)

        def f2(bits):
            match = lax.shift_right_logical(bits, 19) == b1
            d = lax.bitwise_and(lax.shift_right_logical(bits, 7),
                                jnp.int32(0xFFF))
            return d, jnp.where(match, ones_i, zeros_i)

        stream_count(s, rb, f2, use_val=True)
        read_and_rezero(NB)
        b2, r3, _c2 = _select(tmp, tmp2, r2, NB)
        pref = b1 * 4096 + b2

        def f3(bits):
            match = lax.shift_right_logical(bits, 7) == pref
            d = lax.bitwise_and(bits, jnp.int32(0x7F))
            return d, jnp.where(match, ones_i, zeros_i)

        stream_count(s, rb, f3, use_val=True)
        read_and_rezero(NB_C)
        b3, _r4, _c3 = _select(tmp, tmp2, r3, NB_C)
        results.append(pref * 128 + b3)

        if local + 1 < SPT:
            lax.fori_loop(0, CH // L, ones_fill, 0)

    q0, q1 = results
    row_i = jnp.where(lane == 0, jnp.full((L,), q0, jnp.int32),
                      jnp.where(lane == 1, jnp.full((L,), q1, jnp.int32),
                                jnp.zeros((L,), jnp.int32)))
    outbuf_i[...] = row_i
    pltpu.sync_copy(outbuf_i, qbits_hbm.at[wid])


@jax.jit
def kernel(predicted, target, mask):
    pred3 = predicted.reshape(B, H, W)
    tgt3 = target.reshape(B, H, W)
    mask3 = mask.reshape(B, H, W)

    loss, sums = _tc_loss(pred3, tgt3, mask3)

    mesh = plsc.VectorSubcoreMesh(core_axis_name="c", subcore_axis_name="s",
                                  num_cores=NC, num_subcores=NS)
    qbits = pl.kernel(
        _sc_body,
        out_type=jax.ShapeDtypeStruct((NW, L), jnp.int32),
        mesh=mesh,
        compiler_params=pltpu.CompilerParams(needs_layout_passes=False),
        scratch_types=[
            pltpu.VMEM((RPC, W), jnp.float32),
            pltpu.VMEM((RPC, W), jnp.float32),
            pltpu.VMEM((CH,), jnp.int32),
            pltpu.VMEM((CH,), jnp.int32),
            pltpu.VMEM((CH,), jnp.int32),
            pltpu.VMEM((CH,), jnp.int32),
            pltpu.VMEM((NB,), jnp.int32),
            pltpu.VMEM((NB // L,), jnp.int32),
            pltpu.VMEM((NB,), jnp.int32),
            pltpu.VMEM((L,), jnp.int32),
            pltpu.VMEM_SHARED((NS * SPT * NB,), jnp.int32),
            pltpu.SemaphoreType.DMA,
            pltpu.SemaphoreType.DMA,
            pltpu.SemaphoreType.DMA,
            pltpu.SemaphoreType.DMA,
        ],
    )(loss)

    q_loss = lax.bitcast_convert_type(qbits[:, :SPT].reshape(B), jnp.float32)
    wmae = jnp.sum(sums[:, 0, 0]) / (B * N)
    return (q_loss, wmae)


# speedup vs baseline: 1.1774x; 1.0387x over previous
import jax
import jax.numpy as jnp
from jax import lax
from jax.experimental import pallas as pl
from jax.experimental.pallas import tpu as pltpu
from jax.experimental.pallas import tpu_sc as plsc

B = 64
H = 512
W = 512
N = H * W
K = 1 + round(0.01 * 99.0 * (N - 1))

NC = 2
NS = 16
NW = NC * NS
SPT = B // NW

NB = 4096
NB_C = 128
L = 16
CH = 8192
NCH = N // CH
VPC = CH // L
CAP = 32768


def _tc_body(pred_ref, tgt_ref, mask_ref, loss_ref, sums_ref):
    lv = mask_ref[...] * lax.abs(pred_ref[...] - tgt_ref[...])
    loss_ref[...] = lv
    sums_ref[...] = jnp.full((1, 1, 128), jnp.sum(lv), jnp.float32)


def _tc_loss(pred, tgt, mask):
    return pl.pallas_call(
        _tc_body,
        grid=(B,),
        in_specs=[
            pl.BlockSpec((1, H, W), lambda b: (b, 0, 0)),
            pl.BlockSpec((1, H, W), lambda b: (b, 0, 0)),
            pl.BlockSpec((1, H, W), lambda b: (b, 0, 0)),
        ],
        out_specs=[
            pl.BlockSpec((1, H, W), lambda b: (b, 0, 0)),
            pl.BlockSpec((1, 1, 128), lambda b: (b, 0, 0)),
        ],
        out_shape=[
            jax.ShapeDtypeStruct((B, H, W), jnp.float32),
            jax.ShapeDtypeStruct((B, 1, 128), jnp.float32),
        ],
    )(pred, tgt, mask)


def _zero_hist(hist, nb):
    def body(i, _):
        for bank in range(L):
            hist[pl.ds(bank * NB + i * L, L)] = jnp.zeros((L,), jnp.int32)
        return 0
    lax.fori_loop(0, nb // L, body, 0)


def _select(hist, r, nb):
    zero = jnp.int32(0)

    def body(j, carry):
        cum, bstar, cumbef, cnt = carry
        v = jnp.zeros((L,), jnp.int32)
        for bank in range(L):
            v = v + hist[pl.ds(bank * NB + j * L, L)]
        cv = plsc.cumsum(v) + cum
        mlt = cv < r
        msel = jnp.logical_and(cv >= r, (cv - v) < r)
        ones_v = jnp.ones((L,), jnp.int32)
        zeros_v = jnp.zeros((L,), jnp.int32)
        bstar = bstar + jnp.sum(jnp.where(mlt, ones_v, zeros_v))
        cumbef = cumbef + jnp.sum(jnp.where(mlt, v, zeros_v))
        cnt = cnt + jnp.sum(jnp.where(msel, v, zeros_v))
        cum = cum + jnp.sum(v)
        return (cum, bstar, cumbef, cnt)

    cum, bstar, cumbef, cnt = lax.fori_loop(
        0, nb // L, body, (zero, zero, zero, zero))
    return bstar, r - cumbef, cnt


def _stream_pass(src_hbm, s, buf0, buf1, sem0, sem1, chunk_fn, init_carry):
    def start(c, buf, sem):
        off = pl.multiple_of(c * CH, CH)
        pltpu.async_copy(src_hbm.at[s, pl.ds(off, CH)], buf, sem)

    def wait(c, buf, sem):
        off = pl.multiple_of(c * CH, CH)
        pltpu.make_async_copy(src_hbm.at[s, pl.ds(off, CH)], buf, sem).wait()

    start(0, buf0, sem0)
    start(1, buf1, sem1)

    def body(c2, carry):
        c0 = c2 * 2
        wait(c0, buf0, sem0)
        carry = chunk_fn(buf0, c0, carry)

        @pl.when(c0 + 2 < NCH)
        def _():
            start(c0 + 2, buf0, sem0)

        wait(c0 + 1, buf1, sem1)
        carry = chunk_fn(buf1, c0 + 1, carry)

        @pl.when(c0 + 3 < NCH)
        def _():
            start(c0 + 3, buf1, sem1)

        return carry

    return lax.fori_loop(0, NCH // 2, body, init_carry)


def _sc_body(loss_hbm, qbits_hbm,
             buf0, buf1, cand, hist, outbuf_i, sem0, sem1):
    wid = lax.axis_index("s") * NC + lax.axis_index("c")
    lane = lax.broadcasted_iota(jnp.int32, (L,), 0)
    ones_i = jnp.ones((L,), jnp.int32)

    results = []
    for local in range(SPT):
        s = wid * SPT + local

        _zero_hist(hist, NB)

        def chunk_a(buf, c, carry):
            def vbody(i, _):
                bits = lax.bitcast_convert_type(buf[pl.ds(i * L, L)],
                                                jnp.int32)
                d = lax.shift_right_logical(bits, 19)
                plsc.addupdate_scatter(hist, [lane * NB + d], ones_i)
                return 0
            lax.fori_loop(0, VPC, vbody, 0)
            return carry

        _stream_pass(loss_hbm, s, buf0, buf1, sem0, sem1, chunk_a, 0)
        b1, r2, cnt1 = _select(hist, jnp.int32(K), NB)
        docap = cnt1 <= CAP

        _zero_hist(hist, NB)

        def chunk_b(buf, c, cnt):
            def vbody(i, cnt):
                bits = lax.bitcast_convert_type(buf[pl.ds(i * L, L)],
                                                jnp.int32)
                match = lax.shift_right_logical(bits, 19) == b1
                d = lax.bitwise_and(lax.shift_right_logical(bits, 7),
                                    jnp.int32(0xFFF))
                plsc.addupdate_scatter(hist, [lane * NB + d], ones_i,
                                       mask=match)

                @pl.when(docap)
                def _():
                    plsc.store_compressed(cand.at[pl.ds(cnt, L)], bits,
                                          mask=match)

                return cnt + jnp.sum(jnp.where(match, ones_i,
                                               jnp.zeros((L,), jnp.int32)))
            return lax.fori_loop(0, VPC, vbody, cnt)

        _stream_pass(loss_hbm, s, buf0, buf1, sem0, sem1, chunk_b,
                     jnp.int32(0))
        b2, r3, _cnt2 = _select(hist, r2, NB)
        prefix24 = b1 * 4096 + b2

        _zero_hist(hist, NB_C)

        @pl.when(docap)
        def _():
            nv = (cnt1 + (L - 1)) // L

            def vbody(i, _):
                bits = cand[pl.ds(i * L, L)]
                inb = (i * L + lane) < cnt1
                match = jnp.logical_and(
                    lax.shift_right_logical(bits, 7) == prefix24, inb)
                d = lax.bitwise_and(bits, jnp.int32(0x7F))
                plsc.addupdate_scatter(hist, [lane * NB + d], ones_i,
                                       mask=match)
                return 0

            lax.fori_loop(0, nv, vbody, 0)

        @pl.when(jnp.logical_not(docap))
        def _():
            def chunk_c(buf, c, carry):
                def vbody(i, _):
                    bits = lax.bitcast_convert_type(buf[pl.ds(i * L, L)],
                                                    jnp.int32)
                    match = lax.shift_right_logical(bits, 7) == prefix24
                    d = lax.bitwise_and(bits, jnp.int32(0x7F))
                    plsc.addupdate_scatter(hist, [lane * NB + d], ones_i,
                                           mask=match)
                    return 0
                lax.fori_loop(0, VPC, vbody, 0)
                return carry

            _stream_pass(loss_hbm, s, buf0, buf1, sem0, sem1, chunk_c, 0)

        b3, _r4, _c4 = _select(hist, r3, NB_C)
        qbits = prefix24 * 128 + b3
        results.append(qbits)

    q0, q1 = results
    row_i = jnp.where(lane == 0, jnp.full((L,), q0, jnp.int32),
                      jnp.where(lane == 1, jnp.full((L,), q1, jnp.int32),
                                jnp.zeros((L,), jnp.int32)))
    outbuf_i[...] = row_i
    pltpu.sync_copy(outbuf_i, qbits_hbm.at[wid])


@jax.jit
def kernel(predicted, target, mask):
    pred3 = predicted.reshape(B, H, W)
    tgt3 = target.reshape(B, H, W)
    mask3 = mask.reshape(B, H, W)

    loss, sums = _tc_loss(pred3, tgt3, mask3)

    mesh = plsc.VectorSubcoreMesh(core_axis_name="c", subcore_axis_name="s",
                                  num_cores=NC, num_subcores=NS)
    qbits = pl.kernel(
        _sc_body,
        out_type=jax.ShapeDtypeStruct((NW, L), jnp.int32),
        mesh=mesh,
        compiler_params=pltpu.CompilerParams(needs_layout_passes=False),
        scratch_types=[
            pltpu.VMEM((CH,), jnp.float32),
            pltpu.VMEM((CH,), jnp.float32),
            pltpu.VMEM((CAP + L,), jnp.int32),
            pltpu.VMEM((NB * L,), jnp.int32),
            pltpu.VMEM((L,), jnp.int32),
            pltpu.SemaphoreType.DMA,
            pltpu.SemaphoreType.DMA,
        ],
    )(loss.reshape(B, N))

    q_loss = lax.bitcast_convert_type(qbits[:, :SPT].reshape(B), jnp.float32)
    wmae = jnp.sum(sums[:, 0, 0]) / (B * N)
    return (q_loss, wmae)
